# baseline (speedup 1.0000x reference)
import functools

import jax
import jax.numpy as jnp
from jax import lax
from jax.experimental import pallas as pl
from jax.experimental.pallas import tpu as pltpu
from jax.experimental.pallas import tpu_sc as plsc

N = 10000
E = 320000
D = 128
HID = 64

NC = 2
NS = 16
NW = NC * NS
CHUNK = 128
N_PAD = 10240
ROWS_PER_TILE = N_PAD // NS
E_PAD = 327680
CPT = E_PAD // NW // CHUNK
TRASH = N_PAD - 1

RB = 1000
GRID = N // RB

_MESH = plsc.VectorSubcoreMesh(core_axis_name="c", subcore_axis_name="s")


def _unpack_chunk(pk_v, j, src_r, dst_r):
    for c in range(CHUNK // 16):
        p = pk_v[j, pl.ds(c * 16, 16)]
        if src_r is not None:
            src_r[pl.ds(c * 16, 16)] = lax.shift_right_logical(p, 16)
        dst_r[pl.ds(c * 16, 16)] = lax.bitwise_and(p, 0xFFFF)


def _make_degree_kernel():
    @functools.partial(
        pl.kernel,
        out_type=jax.ShapeDtypeStruct((NC, N_PAD, 16), jnp.float32),
        mesh=_MESH,
        scratch_types=[
            pltpu.VMEM((CPT, CHUNK), jnp.int32),
            pltpu.VMEM((CHUNK,), jnp.int32),
            pltpu.VMEM((CHUNK, 16), jnp.float32),
            pltpu.VMEM((CHUNK, 16), jnp.float32),
            pltpu.VMEM_SHARED((N_PAD, 16), jnp.float32),
        ],
        compiler_params=pltpu.CompilerParams(use_tc_tiling_on_sc=False),
    )
    def deg_kernel(pidx, ones, zeros, out, pk_v, dst_r, ones_v, zbuf, acc):
        cid = lax.axis_index("c")
        sid = lax.axis_index("s")
        wid = sid * NC + cid
        r0 = sid * ROWS_PER_TILE
        pltpu.sync_copy(zeros, zbuf)
        for r in range(ROWS_PER_TILE // CHUNK):
            pltpu.sync_copy(zbuf, acc.at[pl.ds(r0 + r * CHUNK, CHUNK)])
        pltpu.sync_copy(ones, ones_v)
        pltpu.sync_copy(pidx.at[pl.ds(wid * CPT, CPT)], pk_v)
        plsc.subcore_barrier()

        def body(j, carry):
            _unpack_chunk(pk_v, j, None, dst_r)
            pltpu.sync_copy(ones_v, acc.at[dst_r], add=True)
            return carry

        lax.fori_loop(0, CPT, body, 0)
        plsc.subcore_barrier()
        for r in range(ROWS_PER_TILE // CHUNK):
            row = r0 + r * CHUNK
            pltpu.sync_copy(acc.at[pl.ds(row, CHUNK)], zbuf)
            for c in range(NC):
                @pl.when(cid == c)
                def _():
                    pltpu.sync_copy(zbuf, out.at[c, pl.ds(row, CHUNK)])

    return deg_kernel


def _make_scatter_kernel(w, chunk, n0, n1, nbuf=8):
    assert (n0 + n1) * NS * chunk == E_PAD
    di = nbuf // 2
    assert n0 % nbuf == 0 and n1 % nbuf == 0
    assert n0 >= nbuf and n1 >= nbuf
    nmax = max(n0, n1)
    copy_chunk = chunk

    @functools.partial(
        pl.kernel,
        out_type=jax.ShapeDtypeStruct((NC, N_PAD, w), jnp.float32),
        mesh=_MESH,
        scratch_types=(
            [pltpu.VMEM((nmax, chunk), jnp.int32)]
            + [pltpu.VMEM((chunk,), jnp.int32)] * (2 * nbuf)
            + [pltpu.VMEM((chunk, w), jnp.float32)] * nbuf
            + [pltpu.VMEM_SHARED((N_PAD, w), jnp.float32)]
            + [pltpu.SemaphoreType.DMA] * (2 * nbuf)
        ),
        compiler_params=pltpu.CompilerParams(use_tc_tiling_on_sc=False),
    )
    def scat_kernel(table, pidx, zeros, out, pk_v, *rest):
        srcs = rest[0:nbuf]
        dsts = rest[nbuf:2 * nbuf]
        rows = rest[2 * nbuf:3 * nbuf]
        cbuf = rows[0]
        acc = rest[3 * nbuf]
        gsem = rest[3 * nbuf + 1:3 * nbuf + 1 + nbuf]
        ssem = rest[3 * nbuf + 1 + nbuf:3 * nbuf + 1 + 2 * nbuf]
        cid = lax.axis_index("c")
        sid = lax.axis_index("s")
        r0 = sid * ROWS_PER_TILE
        pltpu.sync_copy(zeros, cbuf)
        for r in range(ROWS_PER_TILE // copy_chunk):
            pltpu.sync_copy(cbuf, acc.at[pl.ds(r0 + r * copy_chunk,
                                               copy_chunk)])
        n_my = jnp.where(cid == 0, n0, n1)

        @pl.when(cid == 0)
        def _():
            pltpu.sync_copy(pidx.at[pl.ds(sid * n0, n0)],
                            pk_v.at[pl.ds(0, n0)])

        @pl.when(cid == 1)
        def _():
            pltpu.sync_copy(pidx.at[pl.ds(NS * n0 + sid * n1, n1)],
                            pk_v.at[pl.ds(0, n1)])

        plsc.subcore_barrier()

        def unpack(j, b):
            for c in range(chunk // 16):
                p = pk_v[j, pl.ds(c * 16, 16)]
                srcs[b][pl.ds(c * 16, 16)] = lax.shift_right_logical(p, 16)
                dsts[b][pl.ds(c * 16, 16)] = lax.bitwise_and(p, 0xFFFF)

        def start_g(b):
            pltpu.async_copy(table.at[srcs[b]], rows[b], gsem[b])

        def wait_g(b):
            pltpu.make_async_copy(table.at[srcs[b]], rows[b], gsem[b]).wait()

        def start_s(b):
            pltpu.async_copy(rows[b], acc.at[dsts[b]], ssem[b], add=True)

        def wait_s(b):
            pltpu.make_async_copy(rows[b], acc.at[dsts[b]], ssem[b]).wait()

        for j in range(di):
            unpack(j, j)
            start_g(j)
        for j in range(nbuf - di):
            unpack(j + di, j + di)
            start_g(j + di)
            wait_g(j)
            start_s(j)

        def body(grp, carry):
            for b_ in range(nbuf):
                j = (nbuf - di) + grp * nbuf + b_
                bf = b_
                b = (b_ + nbuf - di) % nbuf
                wait_s(bf)
                unpack(j + di, bf)
                start_g(bf)
                wait_g(b)
                start_s(b)
            return carry

        lax.fori_loop(0, (n_my - nbuf) // nbuf, body, 0)
        for k in range(di):
            j = n_my - di + k
            b = (nbuf - di + k) % nbuf
            wait_g(b)
            start_s(b)
        for b in range(nbuf):
            wait_s(b)
        plsc.subcore_barrier()
        for r in range(ROWS_PER_TILE // copy_chunk):
            row = r0 + r * copy_chunk
            pltpu.sync_copy(acc.at[pl.ds(row, copy_chunk)], cbuf)
            for c in range(NC):
                @pl.when(cid == c)
                def _():
                    pltpu.sync_copy(cbuf, out.at[c, pl.ds(row, copy_chunk)])

    return scat_kernel


_DEG = _make_degree_kernel()
_CHUNK_OF = {64: 128, 128: 32}
_SCAT = {64: _make_scatter_kernel(64, 128, 40, 120),
         128: _make_scatter_kernel(128, 32, 160, 480)}


def _dinv_of(degacc):
    def body(d0, d1, o):
        d = d0[0] + d1[0]
        o[...] = lax.rsqrt(d[:, :1] + 1.0)

    return pl.pallas_call(
        body,
        grid=(GRID,),
        in_specs=[pl.BlockSpec((1, RB, 16), lambda i: (0, i, 0)),
                  pl.BlockSpec((1, RB, 16), lambda i: (1, i, 0))],
        out_specs=pl.BlockSpec((RB, 1), lambda i: (i, 0)),
        out_shape=jax.ShapeDtypeStruct((N, 1), jnp.float32),
    )(degacc, degacc)


def _s1(x, w1, dinv):
    def body(x_r, w_r, di, o):
        o[...] = jnp.dot(x_r[...], w_r[...],
                         preferred_element_type=jnp.float32) * di[...]

    return pl.pallas_call(
        body,
        grid=(GRID,),
        in_specs=[pl.BlockSpec((RB, D), lambda i: (i, 0)),
                  pl.BlockSpec((D, HID), lambda i: (0, 0)),
                  pl.BlockSpec((RB, 1), lambda i: (i, 0))],
        out_specs=pl.BlockSpec((RB, HID), lambda i: (i, 0)),
        out_shape=jax.ShapeDtypeStruct((N, HID), jnp.float32),
    )(x, w1, dinv)


def _stage(y, t, b, wn, dinv, relu, win, wout):
    def body(y0, y1, t_r, b_r, w_r, di, o):
        act = di[...] * (y0[0] + y1[0] + t_r[...]) + b_r[...]
        if relu:
            act = jnp.maximum(act, 0.0)
        o[...] = jnp.dot(act, w_r[...],
                         preferred_element_type=jnp.float32) * di[...]

    return pl.pallas_call(
        body,
        grid=(GRID,),
        in_specs=[pl.BlockSpec((1, RB, win), lambda i: (0, i, 0)),
                  pl.BlockSpec((1, RB, win), lambda i: (1, i, 0)),
                  pl.BlockSpec((RB, win), lambda i: (i, 0)),
                  pl.BlockSpec((1, win), lambda i: (0, 0)),
                  pl.BlockSpec((win, wout), lambda i: (0, 0)),
                  pl.BlockSpec((RB, 1), lambda i: (i, 0))],
        out_specs=pl.BlockSpec((RB, wout), lambda i: (i, 0)),
        out_shape=jax.ShapeDtypeStruct((N, wout), jnp.float32),
    )(y, y, t, b.reshape(1, win), wn, dinv)


def _s3(y, t, b2, att_w1, str_w1, dinv):
    def body(y0, y1, t_r, b_r, wa, ws, di, o3, o5):
        h = di[...] * (y0[0] + y1[0] + t_r[...]) + b_r[...]
        o3[...] = jnp.dot(h, wa[...], preferred_element_type=jnp.float32) * di[...]
        o5[...] = jnp.dot(h, ws[...], preferred_element_type=jnp.float32) * di[...]

    return pl.pallas_call(
        body,
        grid=(GRID,),
        in_specs=[pl.BlockSpec((1, RB, HID), lambda i: (0, i, 0)),
                  pl.BlockSpec((1, RB, HID), lambda i: (1, i, 0)),
                  pl.BlockSpec((RB, HID), lambda i: (i, 0)),
                  pl.BlockSpec((1, HID), lambda i: (0, 0)),
                  pl.BlockSpec((HID, HID), lambda i: (0, 0)),
                  pl.BlockSpec((HID, D), lambda i: (0, 0)),
                  pl.BlockSpec((RB, 1), lambda i: (i, 0))],
        out_specs=[pl.BlockSpec((RB, HID), lambda i: (i, 0)),
                   pl.BlockSpec((RB, D), lambda i: (i, 0))],
        out_shape=[jax.ShapeDtypeStruct((N, HID), jnp.float32),
                   jax.ShapeDtypeStruct((N, D), jnp.float32)],
    )(y, y, t, b2.reshape(1, HID), att_w1, str_w1, dinv)


def _s4(y3, t3, att_b1, att_w2, y5, t5, str_b1, dinv):
    def body(y30, y31, t3_r, ab1, wa2, y50, y51, t5_r, sb1, di, o_t4, o_h):
        x1 = jnp.maximum(di[...] * (y30[0] + y31[0] + t3_r[...]) + ab1[...], 0.0)
        o_t4[...] = jnp.dot(x1, wa2[...],
                            preferred_element_type=jnp.float32) * di[...]
        o_h[...] = di[...] * (y50[0] + y51[0] + t5_r[...]) + sb1[...]

    return pl.pallas_call(
        body,
        grid=(GRID,),
        in_specs=[pl.BlockSpec((1, RB, HID), lambda i: (0, i, 0)),
                  pl.BlockSpec((1, RB, HID), lambda i: (1, i, 0)),
                  pl.BlockSpec((RB, HID), lambda i: (i, 0)),
                  pl.BlockSpec((1, HID), lambda i: (0, 0)),
                  pl.BlockSpec((HID, D), lambda i: (0, 0)),
                  pl.BlockSpec((1, RB, D), lambda i: (0, i, 0)),
                  pl.BlockSpec((1, RB, D), lambda i: (1, i, 0)),
                  pl.BlockSpec((RB, D), lambda i: (i, 0)),
                  pl.BlockSpec((1, D), lambda i: (0, 0)),
                  pl.BlockSpec((RB, 1), lambda i: (i, 0))],
        out_specs=[pl.BlockSpec((RB, D), lambda i: (i, 0)),
                   pl.BlockSpec((RB, D), lambda i: (i, 0))],
        out_shape=[jax.ShapeDtypeStruct((N, D), jnp.float32),
                   jax.ShapeDtypeStruct((N, D), jnp.float32)],
    )(y3, y3, t3, att_b1.reshape(1, HID), att_w2,
      y5, y5, t5, str_b1.reshape(1, D), dinv)


def _s5(y4, t4, att_b2, dinv):
    def body(y40, y41, t_r, b_r, di, o):
        o[...] = di[...] * (y40[0] + y41[0] + t_r[...]) + b_r[...]

    return pl.pallas_call(
        body,
        grid=(GRID,),
        in_specs=[pl.BlockSpec((1, RB, D), lambda i: (0, i, 0)),
                  pl.BlockSpec((1, RB, D), lambda i: (1, i, 0)),
                  pl.BlockSpec((RB, D), lambda i: (i, 0)),
                  pl.BlockSpec((1, D), lambda i: (0, 0)),
                  pl.BlockSpec((RB, 1), lambda i: (i, 0))],
        out_specs=pl.BlockSpec((RB, D), lambda i: (i, 0)),
        out_shape=jax.ShapeDtypeStruct((N, D), jnp.float32),
    )(y4, y4, t4, att_b2.reshape(1, D), dinv)


def _gram(h):
    gb = 1024
    ng = (N + gb - 1) // gb

    def body(a, b, o):
        o[...] = lax.dot_general(a[...], b[...], (((1,), (1,)), ((), ())),
                                 preferred_element_type=jnp.float32)

    return pl.pallas_call(
        body,
        grid=(ng, ng),
        in_specs=[pl.BlockSpec((gb, D), lambda i, j: (i, 0)),
                  pl.BlockSpec((gb, D), lambda i, j: (j, 0))],
        out_specs=pl.BlockSpec((gb, gb), lambda i, j: (i, j)),
        out_shape=jax.ShapeDtypeStruct((N, N), jnp.float32),
    )(h, h)


def kernel(x, edge_index, enc_W1, enc_b1, enc_W2, enc_b2,
           att_W1, att_b1, att_W2, att_b2, str_W1, str_b1):
    ei = edge_index.astype(jnp.int32)
    n_pad_e = E_PAD - E
    packed = jnp.concatenate(
        [(ei[0] << 16) | ei[1],
         jnp.full((n_pad_e,), TRASH, jnp.int32)])
    packed128 = packed.reshape(E_PAD // 128, 128)
    packed32 = packed.reshape(E_PAD // 32, 32)
    z16 = jnp.zeros((CHUNK, 16), jnp.float32)
    z64 = jnp.zeros((_CHUNK_OF[64], HID), jnp.float32)
    z128 = jnp.zeros((_CHUNK_OF[128], D), jnp.float32)
    ones16 = jnp.ones((CHUNK, 16), jnp.float32)

    degacc = _DEG(packed128, ones16, z16)
    dinv = _dinv_of(degacc)

    t1 = _s1(x, enc_W1, dinv)
    y1 = _SCAT[64](t1, packed128, z64)
    t2 = _stage(y1, t1, enc_b1, enc_W2, dinv, True, HID, HID)
    y2 = _SCAT[64](t2, packed128, z64)
    t3, t5 = _s3(y2, t2, enc_b2, att_W1, str_W1, dinv)
    y3 = _SCAT[64](t3, packed128, z64)
    y5 = _SCAT[128](t5, packed32, z128)
    t4, h_ = _s4(y3, t3, att_b1, att_W2, y5, t5, str_b1, dinv)
    y4 = _SCAT[128](t4, packed32, z128)
    x_ = _s5(y4, t4, att_b2, dinv)
    s_ = _gram(h_)
    return (x_, s_)


# speedup vs baseline: 1.5507x; 1.5507x over previous
import functools

import jax
import jax.numpy as jnp
from jax import lax
from jax.experimental import pallas as pl
from jax.experimental.pallas import tpu as pltpu
from jax.experimental.pallas import tpu_sc as plsc

N = 10000
E = 320000
D = 128
HID = 64

NC = 2
NS = 16
NW = NC * NS
CHUNK = 128
N_PAD = 10240
ROWS_PER_TILE = N_PAD // NS
E_PAD = 327680
CPT = E_PAD // NW // CHUNK
TRASH = N_PAD - 1

RB = 2000
GRID = N // RB

_MESH = plsc.VectorSubcoreMesh(core_axis_name="c", subcore_axis_name="s")


def _unpack_chunk(pk_v, j, src_r, dst_r):
    for c in range(CHUNK // 16):
        p = pk_v[j, pl.ds(c * 16, 16)]
        if src_r is not None:
            src_r[pl.ds(c * 16, 16)] = lax.shift_right_logical(p, 16)
        dst_r[pl.ds(c * 16, 16)] = lax.bitwise_and(p, 0xFFFF)


def _make_degree_kernel():
    @functools.partial(
        pl.kernel,
        out_type=jax.ShapeDtypeStruct((NC, N_PAD, 16), jnp.float32),
        mesh=_MESH,
        scratch_types=[
            pltpu.VMEM((CPT, CHUNK), jnp.int32),
            pltpu.VMEM((CHUNK,), jnp.int32),
            pltpu.VMEM((CHUNK, 16), jnp.float32),
            pltpu.VMEM((CHUNK, 16), jnp.float32),
            pltpu.VMEM_SHARED((N_PAD, 16), jnp.float32),
        ],
        compiler_params=pltpu.CompilerParams(use_tc_tiling_on_sc=False),
    )
    def deg_kernel(pidx, ones, zeros, out, pk_v, dst_r, ones_v, zbuf, acc):
        cid = lax.axis_index("c")
        sid = lax.axis_index("s")
        wid = sid * NC + cid
        r0 = sid * ROWS_PER_TILE
        pltpu.sync_copy(zeros, zbuf)
        for r in range(ROWS_PER_TILE // CHUNK):
            pltpu.sync_copy(zbuf, acc.at[pl.ds(r0 + r * CHUNK, CHUNK)])
        pltpu.sync_copy(ones, ones_v)
        pltpu.sync_copy(pidx.at[pl.ds(wid * CPT, CPT)], pk_v)
        plsc.subcore_barrier()

        def body(j, carry):
            _unpack_chunk(pk_v, j, None, dst_r)
            pltpu.sync_copy(ones_v, acc.at[dst_r], add=True)
            return carry

        lax.fori_loop(0, CPT, body, 0)
        plsc.subcore_barrier()
        for r in range(ROWS_PER_TILE // CHUNK):
            row = r0 + r * CHUNK
            pltpu.sync_copy(acc.at[pl.ds(row, CHUNK)], zbuf)
            for c in range(NC):
                @pl.when(cid == c)
                def _():
                    pltpu.sync_copy(zbuf, out.at[c, pl.ds(row, CHUNK)])

    return deg_kernel


def _make_scatter_kernel(w, chunk, n0, n1, nbuf=4):
    assert (n0 + n1) * NS * chunk == E_PAD
    di = nbuf // 2
    assert n0 % nbuf == 0 and n1 % nbuf == 0
    assert n0 >= nbuf and n1 >= nbuf
    nmax = max(n0, n1)
    copy_chunk = chunk

    @functools.partial(
        pl.kernel,
        out_type=jax.ShapeDtypeStruct((NC, N_PAD, w), jnp.float32),
        mesh=_MESH,
        scratch_types=(
            [pltpu.VMEM((nmax, chunk), jnp.int32)]
            + [pltpu.VMEM((chunk,), jnp.int32)] * (2 * nbuf)
            + [pltpu.VMEM((chunk, w // 2), jnp.int32)] * nbuf
            + [pltpu.VMEM((chunk, w), jnp.float32)] * nbuf
            + [pltpu.VMEM_SHARED((N_PAD, w), jnp.float32)]
            + [pltpu.SemaphoreType.DMA] * (2 * nbuf)
        ),
        compiler_params=pltpu.CompilerParams(use_tc_tiling_on_sc=False),
    )
    def scat_kernel(table, pidx, zeros, out, pk_v, *rest):
        srcs = rest[0:nbuf]
        dsts = rest[nbuf:2 * nbuf]
        rows_bf = rest[2 * nbuf:3 * nbuf]
        rows = rest[3 * nbuf:4 * nbuf]
        cbuf = rows[0]
        acc = rest[4 * nbuf]
        gsem = rest[4 * nbuf + 1:4 * nbuf + 1 + nbuf]
        ssem = rest[4 * nbuf + 1 + nbuf:4 * nbuf + 1 + 2 * nbuf]
        cid = lax.axis_index("c")
        sid = lax.axis_index("s")
        r0 = sid * ROWS_PER_TILE
        pltpu.sync_copy(zeros, cbuf)
        for r in range(ROWS_PER_TILE // copy_chunk):
            pltpu.sync_copy(cbuf, acc.at[pl.ds(r0 + r * copy_chunk,
                                               copy_chunk)])
        n_my = jnp.where(cid == 0, n0, n1)

        @pl.when(cid == 0)
        def _():
            pltpu.sync_copy(pidx.at[pl.ds(sid * n0, n0)],
                            pk_v.at[pl.ds(0, n0)])

        @pl.when(cid == 1)
        def _():
            pltpu.sync_copy(pidx.at[pl.ds(NS * n0 + sid * n1, n1)],
                            pk_v.at[pl.ds(0, n1)])

        plsc.subcore_barrier()

        def unpack(j, b):
            for c in range(chunk // 16):
                p = pk_v[j, pl.ds(c * 16, 16)]
                srcs[b][pl.ds(c * 16, 16)] = lax.shift_right_logical(p, 16)
                dsts[b][pl.ds(c * 16, 16)] = lax.bitwise_and(p, 0xFFFF)

        def start_g(b):
            pltpu.async_copy(table.at[srcs[b]], rows_bf[b], gsem[b])

        def wait_g(b):
            pltpu.make_async_copy(table.at[srcs[b]], rows_bf[b],
                                  gsem[b]).wait()

        def convert(b):
            def cbody(r, carry):
                for c in range(w // 32):
                    wv = rows_bf[b][r, pl.ds(c * 16, 16)]
                    lo = lax.bitcast_convert_type(
                        lax.shift_left(wv, 16), jnp.float32)
                    hi = lax.bitcast_convert_type(
                        lax.bitwise_and(wv, jnp.int32(-65536)), jnp.float32)
                    rows[b][r, pl.ds(c * 32, 16)] = lo
                    rows[b][r, pl.ds(c * 32 + 16, 16)] = hi
                return carry

            lax.fori_loop(0, chunk, cbody, 0)

        def start_s(b):
            pltpu.async_copy(rows[b], acc.at[dsts[b]], ssem[b], add=True)

        def wait_s(b):
            pltpu.make_async_copy(rows[b], acc.at[dsts[b]], ssem[b]).wait()

        for j in range(di):
            unpack(j, j)
            start_g(j)
        for j in range(nbuf - di):
            unpack(j + di, j + di)
            start_g(j + di)
            wait_g(j)
            convert(j)
            start_s(j)

        def body(grp, carry):
            for b_ in range(nbuf):
                j = (nbuf - di) + grp * nbuf + b_
                bf = b_
                b = (b_ + nbuf - di) % nbuf
                wait_s(bf)
                unpack(j + di, bf)
                start_g(bf)
                wait_g(b)
                convert(b)
                start_s(b)
            return carry

        lax.fori_loop(0, (n_my - nbuf) // nbuf, body, 0)
        for k in range(di):
            b = (nbuf - di + k) % nbuf
            wait_g(b)
            convert(b)
            start_s(b)
        for b in range(nbuf):
            wait_s(b)
        plsc.subcore_barrier()
        for r in range(ROWS_PER_TILE // copy_chunk):
            row = r0 + r * copy_chunk
            pltpu.sync_copy(acc.at[pl.ds(row, copy_chunk)], cbuf)
            for c in range(NC):
                @pl.when(cid == c)
                def _():
                    pltpu.sync_copy(cbuf, out.at[c, pl.ds(row, copy_chunk)])

    return scat_kernel


_DEG = _make_degree_kernel()
_CHUNK_OF = {64: 128, 128: 32}
_SCAT = {64: _make_scatter_kernel(64, 128, 80, 80),
         128: _make_scatter_kernel(128, 32, 320, 320)}


def _dinv_of(degacc):
    def body(d0, d1, o):
        d = d0[0] + d1[0]
        o[...] = lax.rsqrt(d[:, :1] + 1.0)

    return pl.pallas_call(
        body,
        grid=(GRID,),
        in_specs=[pl.BlockSpec((1, RB, 16), lambda i: (0, i, 0)),
                  pl.BlockSpec((1, RB, 16), lambda i: (1, i, 0))],
        out_specs=pl.BlockSpec((RB, 1), lambda i: (i, 0)),
        out_shape=jax.ShapeDtypeStruct((N, 1), jnp.float32),
    )(degacc, degacc)


def _s1(x, w1, dinv):
    def body(x_r, w_r, di, o, ob):
        t = jnp.dot(x_r[...], w_r[...],
                    preferred_element_type=jnp.float32) * di[...]
        o[...] = t
        ob[...] = t.astype(jnp.bfloat16)

    return pl.pallas_call(
        body,
        grid=(GRID,),
        in_specs=[pl.BlockSpec((RB, D), lambda i: (i, 0)),
                  pl.BlockSpec((D, HID), lambda i: (0, 0)),
                  pl.BlockSpec((RB, 1), lambda i: (i, 0))],
        out_specs=[pl.BlockSpec((RB, HID), lambda i: (i, 0)),
                   pl.BlockSpec((RB, HID), lambda i: (i, 0))],
        out_shape=[jax.ShapeDtypeStruct((N, HID), jnp.float32),
                   jax.ShapeDtypeStruct((N, HID), jnp.bfloat16)],
    )(x, w1, dinv)


def _stage(y, t, b, wn, dinv, relu, win, wout):
    def body(y0, y1, t_r, b_r, w_r, di, o, ob):
        act = di[...] * (y0[0] + y1[0] + t_r[...]) + b_r[...]
        if relu:
            act = jnp.maximum(act, 0.0)
        t = jnp.dot(act, w_r[...],
                    preferred_element_type=jnp.float32) * di[...]
        o[...] = t
        ob[...] = t.astype(jnp.bfloat16)

    return pl.pallas_call(
        body,
        grid=(GRID,),
        in_specs=[pl.BlockSpec((1, RB, win), lambda i: (0, i, 0)),
                  pl.BlockSpec((1, RB, win), lambda i: (1, i, 0)),
                  pl.BlockSpec((RB, win), lambda i: (i, 0)),
                  pl.BlockSpec((1, win), lambda i: (0, 0)),
                  pl.BlockSpec((win, wout), lambda i: (0, 0)),
                  pl.BlockSpec((RB, 1), lambda i: (i, 0))],
        out_specs=[pl.BlockSpec((RB, wout), lambda i: (i, 0)),
                   pl.BlockSpec((RB, wout), lambda i: (i, 0))],
        out_shape=[jax.ShapeDtypeStruct((N, wout), jnp.float32),
                   jax.ShapeDtypeStruct((N, wout), jnp.bfloat16)],
    )(y, y, t, b.reshape(1, win), wn, dinv)


def _s3(y, t, b2, att_w1, str_w1, dinv):
    def body(y0, y1, t_r, b_r, wa, ws, di, o3, o3b, o5, o5b):
        h = di[...] * (y0[0] + y1[0] + t_r[...]) + b_r[...]
        t3 = jnp.dot(h, wa[...], preferred_element_type=jnp.float32) * di[...]
        t5 = jnp.dot(h, ws[...], preferred_element_type=jnp.float32) * di[...]
        o3[...] = t3
        o3b[...] = t3.astype(jnp.bfloat16)
        o5[...] = t5
        o5b[...] = t5.astype(jnp.bfloat16)

    return pl.pallas_call(
        body,
        grid=(GRID,),
        in_specs=[pl.BlockSpec((1, RB, HID), lambda i: (0, i, 0)),
                  pl.BlockSpec((1, RB, HID), lambda i: (1, i, 0)),
                  pl.BlockSpec((RB, HID), lambda i: (i, 0)),
                  pl.BlockSpec((1, HID), lambda i: (0, 0)),
                  pl.BlockSpec((HID, HID), lambda i: (0, 0)),
                  pl.BlockSpec((HID, D), lambda i: (0, 0)),
                  pl.BlockSpec((RB, 1), lambda i: (i, 0))],
        out_specs=[pl.BlockSpec((RB, HID), lambda i: (i, 0)),
                   pl.BlockSpec((RB, HID), lambda i: (i, 0)),
                   pl.BlockSpec((RB, D), lambda i: (i, 0)),
                   pl.BlockSpec((RB, D), lambda i: (i, 0))],
        out_shape=[jax.ShapeDtypeStruct((N, HID), jnp.float32),
                   jax.ShapeDtypeStruct((N, HID), jnp.bfloat16),
                   jax.ShapeDtypeStruct((N, D), jnp.float32),
                   jax.ShapeDtypeStruct((N, D), jnp.bfloat16)],
    )(y, y, t, b2.reshape(1, HID), att_w1, str_w1, dinv)


def _s4(y3, t3, att_b1, att_w2, y5, t5, str_b1, dinv):
    def body(y30, y31, t3_r, ab1, wa2, y50, y51, t5_r, sb1, di,
             o_t4, o_t4b, o_h):
        x1 = jnp.maximum(di[...] * (y30[0] + y31[0] + t3_r[...]) + ab1[...], 0.0)
        t4 = jnp.dot(x1, wa2[...],
                     preferred_element_type=jnp.float32) * di[...]
        o_t4[...] = t4
        o_t4b[...] = t4.astype(jnp.bfloat16)
        o_h[...] = di[...] * (y50[0] + y51[0] + t5_r[...]) + sb1[...]

    return pl.pallas_call(
        body,
        grid=(GRID,),
        in_specs=[pl.BlockSpec((1, RB, HID), lambda i: (0, i, 0)),
                  pl.BlockSpec((1, RB, HID), lambda i: (1, i, 0)),
                  pl.BlockSpec((RB, HID), lambda i: (i, 0)),
                  pl.BlockSpec((1, HID), lambda i: (0, 0)),
                  pl.BlockSpec((HID, D), lambda i: (0, 0)),
                  pl.BlockSpec((1, RB, D), lambda i: (0, i, 0)),
                  pl.BlockSpec((1, RB, D), lambda i: (1, i, 0)),
                  pl.BlockSpec((RB, D), lambda i: (i, 0)),
                  pl.BlockSpec((1, D), lambda i: (0, 0)),
                  pl.BlockSpec((RB, 1), lambda i: (i, 0))],
        out_specs=[pl.BlockSpec((RB, D), lambda i: (i, 0)),
                   pl.BlockSpec((RB, D), lambda i: (i, 0)),
                   pl.BlockSpec((RB, D), lambda i: (i, 0))],
        out_shape=[jax.ShapeDtypeStruct((N, D), jnp.float32),
                   jax.ShapeDtypeStruct((N, D), jnp.bfloat16),
                   jax.ShapeDtypeStruct((N, D), jnp.float32)],
    )(y3, y3, t3, att_b1.reshape(1, HID), att_w2,
      y5, y5, t5, str_b1.reshape(1, D), dinv)


def _s5(y4, t4, att_b2, dinv):
    def body(y40, y41, t_r, b_r, di, o):
        o[...] = di[...] * (y40[0] + y41[0] + t_r[...]) + b_r[...]

    return pl.pallas_call(
        body,
        grid=(GRID,),
        in_specs=[pl.BlockSpec((1, RB, D), lambda i: (0, i, 0)),
                  pl.BlockSpec((1, RB, D), lambda i: (1, i, 0)),
                  pl.BlockSpec((RB, D), lambda i: (i, 0)),
                  pl.BlockSpec((1, D), lambda i: (0, 0)),
                  pl.BlockSpec((RB, 1), lambda i: (i, 0))],
        out_specs=pl.BlockSpec((RB, D), lambda i: (i, 0)),
        out_shape=jax.ShapeDtypeStruct((N, D), jnp.float32),
    )(y4, y4, t4, att_b2.reshape(1, D), dinv)


def _gram(h):
    gb = 1024
    ng = (N + gb - 1) // gb

    def body(a, b, o):
        o[...] = lax.dot_general(a[...], b[...], (((1,), (1,)), ((), ())),
                                 preferred_element_type=jnp.float32)

    return pl.pallas_call(
        body,
        grid=(ng, ng),
        in_specs=[pl.BlockSpec((gb, D), lambda i, j: (i, 0)),
                  pl.BlockSpec((gb, D), lambda i, j: (j, 0))],
        out_specs=pl.BlockSpec((gb, gb), lambda i, j: (i, j)),
        out_shape=jax.ShapeDtypeStruct((N, N), jnp.float32),
    )(h, h)


def _tbl(tb):
    w = tb.shape[1]
    x = tb.reshape(N, w // 32, 2, 16).swapaxes(2, 3).reshape(N, w // 2, 2)
    return lax.bitcast_convert_type(x, jnp.int32)


def kernel(x, edge_index, enc_W1, enc_b1, enc_W2, enc_b2,
           att_W1, att_b1, att_W2, att_b2, str_W1, str_b1):
    ei = edge_index.astype(jnp.int32)
    n_pad_e = E_PAD - E
    packed = jnp.concatenate(
        [(ei[0] << 16) | ei[1],
         jnp.full((n_pad_e,), TRASH, jnp.int32)])
    packed128 = packed.reshape(E_PAD // 128, 128)
    packed32 = packed.reshape(E_PAD // 32, 32)
    z16 = jnp.zeros((CHUNK, 16), jnp.float32)
    z64 = jnp.zeros((_CHUNK_OF[64], HID), jnp.float32)
    z128 = jnp.zeros((_CHUNK_OF[128], D), jnp.float32)
    ones16 = jnp.ones((CHUNK, 16), jnp.float32)

    degacc = _DEG(packed128, ones16, z16)
    dinv = _dinv_of(degacc)

    t1, t1b = _s1(x, enc_W1, dinv)
    y1 = _SCAT[64](_tbl(t1b), packed128, z64)
    t2, t2b = _stage(y1, t1, enc_b1, enc_W2, dinv, True, HID, HID)
    y2 = _SCAT[64](_tbl(t2b), packed128, z64)
    t3, t3b, t5, t5b = _s3(y2, t2, enc_b2, att_W1, str_W1, dinv)
    y3 = _SCAT[64](_tbl(t3b), packed128, z64)
    y5 = _SCAT[128](_tbl(t5b), packed32, z128)
    t4, t4b, h_ = _s4(y3, t3, att_b1, att_W2, y5, t5, str_b1, dinv)
    y4 = _SCAT[128](_tbl(t4b), packed32, z128)
    x_ = _s5(y4, t4, att_b2, dinv)
    s_ = _gram(h_)
    return (x_, s_)
